# baseline (speedup 1.0000x reference)
import functools

import jax
import jax.numpy as jnp
from jax import lax
from jax.experimental import pallas as pl
from jax.experimental.pallas import tpu as pltpu
from jax.experimental.pallas import tpu_sc as plsc

NC = 2
NS = 16
NW = NC * NS
L = 16

CT = 512
CHUNK = 512
NBUF = 5


@functools.lru_cache(maxsize=None)
def _make_transpose(dim: int, vocab: int):
    n_full = vocab // CT
    rem = vocab - n_full * CT
    mesh = plsc.VectorSubcoreMesh(core_axis_name="c", subcore_axis_name="s")

    @functools.partial(
        pl.kernel,
        mesh=mesh,
        compiler_params=pltpu.CompilerParams(needs_layout_passes=False),
        out_type=jax.ShapeDtypeStruct((vocab * dim,), jnp.float32),
        scratch_types=[
            pltpu.VMEM((dim, CT), jnp.float32),
            pltpu.VMEM((CT * dim,), jnp.float32),
        ],
    )
    def transpose_kernel(tablet_hbm, tail_hbm, out_hbm, va, vb):
        wid = lax.axis_index("s") * NC + lax.axis_index("c")
        lane32 = lax.iota(jnp.int32, L) * dim

        def body(k, carry):
            c0 = pl.multiple_of((k * NW + wid) * CT, CT)
            pltpu.sync_copy(tablet_hbm.at[:, pl.ds(c0, CT)], va)
            for d in range(dim):
                for j in range(CT // L):
                    v = va[d, pl.ds(j * L, L)]
                    plsc.store_scatter(vb, [lane32 + (j * (L * dim) + d)], v)
            pltpu.sync_copy(vb, out_hbm.at[pl.ds(c0 * dim, CT * dim)])
            return carry

        n_w = (n_full - wid + NW - 1) // NW
        lax.fori_loop(0, n_w, body, 0)

        if rem:
            @pl.when(wid == 0)
            def _tail():
                pltpu.sync_copy(tail_hbm, va.at[:, pl.ds(0, 128)])
                for d in range(dim):
                    def tb(j, cc):
                        v = va[d, pl.ds(j * L, L)]
                        plsc.store_scatter(vb, [lane32 + (j * (L * dim) + d)], v)
                        return cc

                    lax.fori_loop(0, rem // L, tb, 0)
                pltpu.sync_copy(
                    vb.at[pl.ds(0, rem * dim)],
                    out_hbm.at[pl.ds((vocab - rem) * dim, rem * dim)],
                )

    return transpose_kernel


@functools.lru_cache(maxsize=None)
def _make_gather(n_flat: int, dim: int):
    assert n_flat % NW == 0
    per_w = n_flat // NW
    assert per_w % (CHUNK * NBUF) == 0
    n_rounds = per_w // (CHUNK * NBUF)

    mesh = plsc.VectorSubcoreMesh(core_axis_name="c", subcore_axis_name="s")

    @functools.partial(
        pl.kernel,
        mesh=mesh,
        compiler_params=pltpu.CompilerParams(use_tc_tiling_on_sc=False),
        out_type=jax.ShapeDtypeStruct((n_flat, 128), jnp.float32),
        scratch_types=[
            pltpu.VMEM((per_w,), jnp.int32),
            *[pltpu.VMEM((CHUNK, dim), jnp.float32) for _ in range(NBUF)],
            *[pltpu.SemaphoreType.DMA for _ in range(NBUF)],
            *[pltpu.SemaphoreType.DMA for _ in range(NBUF)],
        ],
    )
    def gather_kernel(idx_hbm, table_hbm, out_hbm, idx_v, *bufs_and_sems):
        rows = bufs_and_sems[:NBUF]
        sem_g = bufs_and_sems[NBUF:2 * NBUF]
        sem_w = bufs_and_sems[2 * NBUF:3 * NBUF]

        wid = lax.axis_index("s") * NC + lax.axis_index("c")
        base = wid * per_w
        pltpu.sync_copy(idx_hbm.at[pl.ds(base, per_w)], idx_v)

        def body(i, carry):
            rbase = i * (CHUNK * NBUF)
            for b in range(NBUF):
                off = rbase + b * CHUNK

                @pl.when(i > 0)
                def _wait_prev_write():
                    pltpu.make_async_copy(
                        rows[b],
                        out_hbm.at[pl.ds(base + off, CHUNK), pl.ds(0, dim)],
                        sem_w[b],
                    ).wait()

                pltpu.async_copy(
                    table_hbm.at[idx_v.at[pl.ds(off, CHUNK)]], rows[b], sem_g[b]
                )
            for b in range(NBUF):
                off = rbase + b * CHUNK
                pltpu.make_async_copy(
                    table_hbm.at[idx_v.at[pl.ds(off, CHUNK)]], rows[b], sem_g[b]
                ).wait()
                pltpu.async_copy(
                    rows[b],
                    out_hbm.at[pl.ds(base + off, CHUNK), pl.ds(0, dim)],
                    sem_w[b],
                )
            return carry

        lax.fori_loop(0, n_rounds, body, 0)
        for b in range(NBUF):
            pltpu.make_async_copy(
                rows[b], out_hbm.at[pl.ds(base, CHUNK), pl.ds(0, dim)], sem_w[b]
            ).wait()

    return gather_kernel


def kernel(x, table):
    b, t = x.shape
    vocab, dim = table.shape
    flat = jnp.reshape(x, (-1,)).astype(jnp.int32)
    tablet = jnp.transpose(table)
    rem = vocab % CT
    tail = jnp.pad(tablet[:, vocab - rem:], ((0, 0), (0, 128 - rem)))
    flat_table = _make_transpose(dim, vocab)(tablet, tail)
    tablec = jnp.reshape(flat_table, (vocab, dim))
    out128 = _make_gather(b * t, dim)(flat, tablec)
    return jnp.reshape(out128[:, :dim], (b, t, dim))


# speedup vs baseline: 1.0005x; 1.0005x over previous
import functools

import jax
import jax.numpy as jnp
from jax import lax
from jax.experimental import pallas as pl
from jax.experimental.pallas import tpu as pltpu
from jax.experimental.pallas import tpu_sc as plsc

NC = 2
NS = 16
NW = NC * NS
L = 16

CT = 512
CHUNK = 512
NBUF = 5


@functools.lru_cache(maxsize=None)
def _make_transpose(dim: int, vocab: int):
    n_full = vocab // CT
    rem = vocab - n_full * CT
    mesh = plsc.VectorSubcoreMesh(core_axis_name="c", subcore_axis_name="s")

    @functools.partial(
        pl.kernel,
        mesh=mesh,
        compiler_params=pltpu.CompilerParams(needs_layout_passes=False),
        out_type=jax.ShapeDtypeStruct((vocab * dim,), jnp.float32),
        scratch_types=[
            pltpu.VMEM((dim, CT), jnp.float32),
            pltpu.VMEM((CT * dim,), jnp.float32),
        ],
    )
    def transpose_kernel(tablet_hbm, tail_hbm, out_hbm, va, vb):
        wid = lax.axis_index("s") * NC + lax.axis_index("c")
        lane32 = lax.iota(jnp.int32, L) * dim

        def body(k, carry):
            c0 = pl.multiple_of((k * NW + wid) * CT, CT)
            pltpu.sync_copy(tablet_hbm.at[:, pl.ds(c0, CT)], va)
            for d in range(dim):
                vs = [va[d, pl.ds(j * L, L)] for j in range(CT // L)]
                for j in range(CT // L):
                    plsc.store_scatter(vb, [lane32 + (j * (L * dim) + d)], vs[j])
            pltpu.sync_copy(vb, out_hbm.at[pl.ds(c0 * dim, CT * dim)])
            return carry

        n_w = (n_full - wid + NW - 1) // NW
        lax.fori_loop(0, n_w, body, 0)

        if rem:
            @pl.when(wid == 0)
            def _tail():
                pltpu.sync_copy(tail_hbm, va.at[:, pl.ds(0, 128)])
                for d in range(dim):
                    def tb(j, cc):
                        v = va[d, pl.ds(j * L, L)]
                        plsc.store_scatter(vb, [lane32 + (j * (L * dim) + d)], v)
                        return cc

                    lax.fori_loop(0, rem // L, tb, 0)
                pltpu.sync_copy(
                    vb.at[pl.ds(0, rem * dim)],
                    out_hbm.at[pl.ds((vocab - rem) * dim, rem * dim)],
                )

    return transpose_kernel


@functools.lru_cache(maxsize=None)
def _make_gather(n_flat: int, dim: int):
    assert n_flat % NW == 0
    per_w = n_flat // NW
    assert per_w % (CHUNK * NBUF) == 0
    n_rounds = per_w // (CHUNK * NBUF)

    mesh = plsc.VectorSubcoreMesh(core_axis_name="c", subcore_axis_name="s")

    @functools.partial(
        pl.kernel,
        mesh=mesh,
        compiler_params=pltpu.CompilerParams(use_tc_tiling_on_sc=False),
        out_type=jax.ShapeDtypeStruct((n_flat, 128), jnp.float32),
        scratch_types=[
            pltpu.VMEM((per_w,), jnp.int32),
            *[pltpu.VMEM((CHUNK, dim), jnp.float32) for _ in range(NBUF)],
            *[pltpu.SemaphoreType.DMA for _ in range(NBUF)],
            *[pltpu.SemaphoreType.DMA for _ in range(NBUF)],
        ],
    )
    def gather_kernel(idx_hbm, table_hbm, out_hbm, idx_v, *bufs_and_sems):
        rows = bufs_and_sems[:NBUF]
        sem_g = bufs_and_sems[NBUF:2 * NBUF]
        sem_w = bufs_and_sems[2 * NBUF:3 * NBUF]

        wid = lax.axis_index("s") * NC + lax.axis_index("c")
        base = wid * per_w
        pltpu.sync_copy(idx_hbm.at[pl.ds(base, per_w)], idx_v)

        def body(i, carry):
            rbase = i * (CHUNK * NBUF)
            for b in range(NBUF):
                off = rbase + b * CHUNK

                @pl.when(i > 0)
                def _wait_prev_write():
                    pltpu.make_async_copy(
                        rows[b],
                        out_hbm.at[pl.ds(base + off, CHUNK), pl.ds(0, dim)],
                        sem_w[b],
                    ).wait()

                pltpu.async_copy(
                    table_hbm.at[idx_v.at[pl.ds(off, CHUNK)]], rows[b], sem_g[b]
                )
            for b in range(NBUF):
                off = rbase + b * CHUNK
                pltpu.make_async_copy(
                    table_hbm.at[idx_v.at[pl.ds(off, CHUNK)]], rows[b], sem_g[b]
                ).wait()
                pltpu.async_copy(
                    rows[b],
                    out_hbm.at[pl.ds(base + off, CHUNK), pl.ds(0, dim)],
                    sem_w[b],
                )
            return carry

        lax.fori_loop(0, n_rounds, body, 0)
        for b in range(NBUF):
            pltpu.make_async_copy(
                rows[b], out_hbm.at[pl.ds(base, CHUNK), pl.ds(0, dim)], sem_w[b]
            ).wait()

    return gather_kernel


def kernel(x, table):
    b, t = x.shape
    vocab, dim = table.shape
    flat = jnp.reshape(x, (-1,)).astype(jnp.int32)
    tablet = jnp.transpose(table)
    rem = vocab % CT
    tail = jnp.pad(tablet[:, vocab - rem:], ((0, 0), (0, 128 - rem)))
    flat_table = _make_transpose(dim, vocab)(tablet, tail)
    tablec = jnp.reshape(flat_table, (vocab, dim))
    out128 = _make_gather(b * t, dim)(flat, tablec)
    return jnp.reshape(out128[:, :dim], (b, t, dim))


# speedup vs baseline: 1.1498x; 1.1492x over previous
import functools

import jax
import jax.numpy as jnp
from jax import lax
from jax.experimental import pallas as pl
from jax.experimental.pallas import tpu as pltpu
from jax.experimental.pallas import tpu_sc as plsc

NC = 2
NS = 16
NW = NC * NS
L = 16

CT = 512
CHUNK = 512
NBUF = 5


@functools.lru_cache(maxsize=None)
def _make_transpose(dim: int, vocab: int):
    n_full = (vocab // CT) // NW * NW
    n_w = n_full // NW
    rem = vocab - n_full * CT
    rem_pad = -(-rem // 128) * 128
    assert n_w >= 3 and rem_pad <= 2 * CT
    mesh = plsc.VectorSubcoreMesh(core_axis_name="c", subcore_axis_name="s")

    @functools.partial(
        pl.kernel,
        mesh=mesh,
        compiler_params=pltpu.CompilerParams(needs_layout_passes=False),
        out_type=jax.ShapeDtypeStruct((vocab * dim,), jnp.float32),
        scratch_types=[
            *[pltpu.VMEM((dim, CT), jnp.float32) for _ in range(2)],
            *[pltpu.VMEM((CT * dim,), jnp.float32) for _ in range(2)],
            pltpu.VMEM((dim, rem_pad), jnp.float32),
            pltpu.VMEM((rem_pad * dim,), jnp.float32),
            *[pltpu.SemaphoreType.DMA for _ in range(2)],
            *[pltpu.SemaphoreType.DMA for _ in range(2)],
        ],
    )
    def transpose_kernel(tablet_hbm, tail_hbm, out_hbm,
                         va0, va1, vb0, vb1, vat, vbt, sr0, sr1, sw0, sw1):
        va = (va0, va1, vat)
        vb = (vb0, vb1, vbt)
        sr = (sr0, sr1)
        sw = (sw0, sw1)
        wid = lax.axis_index("s") * NC + lax.axis_index("c")
        lane32 = lax.iota(jnp.int32, L) * dim

        def col0(k):
            return pl.multiple_of((k * NW + wid) * CT, CT)

        def read(k, p):
            pltpu.async_copy(
                tablet_hbm.at[:, pl.ds(col0(k), CT)], va[p], sr[p]
            )

        def wait_read(p):
            pltpu.make_async_copy(
                tablet_hbm.at[:, pl.ds(0, CT)], va[p], sr[p]
            ).wait()

        def compute(p, n_cols, batch=8):
            del batch
            for d in range(dim):
                def jb(j, cc):
                    v = va[p][d, pl.ds(j * L, L)]
                    plsc.store_scatter(
                        vb[p], [lane32 + (j * (L * dim) + d)], v
                    )
                    return cc

                lax.fori_loop(0, n_cols // L, jb, 0)

        def write(k, p):
            pltpu.async_copy(
                vb[p], out_hbm.at[pl.ds(col0(k) * dim, CT * dim)], sw[p]
            )

        def wait_write(p):
            pltpu.make_async_copy(
                vb[p], out_hbm.at[pl.ds(0, CT * dim)], sw[p]
            ).wait()

        read(0, 0)

        def body(i, carry):
            for p in range(2):
                k = 2 * i + p
                wait_read(p)
                read(k + 1, 1 - p)

                @pl.when(i > 0)
                def _ww():
                    wait_write(p)

                compute(p, CT)
                write(k, p)
            return carry

        half = (n_w - 1) // 2
        lax.fori_loop(0, half, body, 0)
        for k in range(2 * half, n_w):
            p = k % 2
            wait_read(p)
            if k + 1 < n_w:
                read(k + 1, 1 - p)
            wait_write(p)
            compute(p, CT)
            write(k, p)
        wait_write(0)
        wait_write(1)

        if rem:
            @pl.when(wid == 0)
            def _tail():
                pltpu.sync_copy(tail_hbm, va[2])
                compute(2, rem, batch=4)
                pltpu.sync_copy(
                    vb[2].at[pl.ds(0, rem * dim)],
                    out_hbm.at[pl.ds((vocab - rem) * dim, rem * dim)],
                )

    return transpose_kernel


@functools.lru_cache(maxsize=None)
def _make_gather(n_flat: int, dim: int):
    assert n_flat % NW == 0
    per_w = n_flat // NW
    assert per_w % (CHUNK * NBUF) == 0
    n_rounds = per_w // (CHUNK * NBUF)

    mesh = plsc.VectorSubcoreMesh(core_axis_name="c", subcore_axis_name="s")

    @functools.partial(
        pl.kernel,
        mesh=mesh,
        compiler_params=pltpu.CompilerParams(use_tc_tiling_on_sc=False),
        out_type=jax.ShapeDtypeStruct((n_flat, 128), jnp.float32),
        scratch_types=[
            pltpu.VMEM((per_w,), jnp.int32),
            *[pltpu.VMEM((CHUNK, dim), jnp.float32) for _ in range(NBUF)],
            *[pltpu.SemaphoreType.DMA for _ in range(NBUF)],
            *[pltpu.SemaphoreType.DMA for _ in range(NBUF)],
        ],
    )
    def gather_kernel(idx_hbm, table_hbm, out_hbm, idx_v, *bufs_and_sems):
        rows = bufs_and_sems[:NBUF]
        sem_g = bufs_and_sems[NBUF:2 * NBUF]
        sem_w = bufs_and_sems[2 * NBUF:3 * NBUF]

        wid = lax.axis_index("s") * NC + lax.axis_index("c")
        base = wid * per_w
        pltpu.sync_copy(idx_hbm.at[pl.ds(base, per_w)], idx_v)

        def body(i, carry):
            rbase = i * (CHUNK * NBUF)
            for b in range(NBUF):
                off = rbase + b * CHUNK

                @pl.when(i > 0)
                def _wait_prev_write():
                    pltpu.make_async_copy(
                        rows[b],
                        out_hbm.at[pl.ds(base + off, CHUNK), pl.ds(0, dim)],
                        sem_w[b],
                    ).wait()

                pltpu.async_copy(
                    table_hbm.at[idx_v.at[pl.ds(off, CHUNK)]], rows[b], sem_g[b]
                )
            for b in range(NBUF):
                off = rbase + b * CHUNK
                pltpu.make_async_copy(
                    table_hbm.at[idx_v.at[pl.ds(off, CHUNK)]], rows[b], sem_g[b]
                ).wait()
                pltpu.async_copy(
                    rows[b],
                    out_hbm.at[pl.ds(base + off, CHUNK), pl.ds(0, dim)],
                    sem_w[b],
                )
            return carry

        lax.fori_loop(0, n_rounds, body, 0)
        for b in range(NBUF):
            pltpu.make_async_copy(
                rows[b], out_hbm.at[pl.ds(base, CHUNK), pl.ds(0, dim)], sem_w[b]
            ).wait()

    return gather_kernel


def kernel(x, table):
    b, t = x.shape
    vocab, dim = table.shape
    flat = jnp.reshape(x, (-1,)).astype(jnp.int32)
    tablet = jnp.transpose(table)
    rem = vocab - (vocab // CT) // NW * NW * CT
    rem_pad = -(-rem // 128) * 128
    tail = jnp.pad(tablet[:, vocab - rem:], ((0, 0), (0, rem_pad - rem)))
    flat_table = _make_transpose(dim, vocab)(tablet, tail)
    tablec = jnp.reshape(flat_table, (vocab, dim))
    out128 = _make_gather(b * t, dim)(flat, tablec)
    return jnp.reshape(out128[:, :dim], (b, t, dim))


# speedup vs baseline: 1.3663x; 1.1883x over previous
import functools

import jax
import jax.numpy as jnp
from jax import lax
from jax.experimental import pallas as pl
from jax.experimental.pallas import tpu as pltpu
from jax.experimental.pallas import tpu_sc as plsc

NC = 2
NS = 16
NW = NC * NS
L = 16

CT = 512
CHUNK = 512
NBUF = 5


@functools.lru_cache(maxsize=None)
def _make_transpose(dim: int, vocab: int):
    n_full = (vocab // CT) // NW * NW
    n_w = n_full // NW
    rem = vocab - n_full * CT
    rem_pad = -(-rem // 128) * 128
    assert n_w >= 3 and rem_pad <= 2 * CT
    mesh = plsc.VectorSubcoreMesh(core_axis_name="c", subcore_axis_name="s")

    @functools.partial(
        pl.kernel,
        mesh=mesh,
        compiler_params=pltpu.CompilerParams(needs_layout_passes=False),
        out_type=jax.ShapeDtypeStruct((vocab * dim,), jnp.float32),
        scratch_types=[
            *[pltpu.VMEM((dim, CT), jnp.float32) for _ in range(2)],
            *[pltpu.VMEM((CT * dim,), jnp.float32) for _ in range(2)],
            pltpu.VMEM((dim, rem_pad), jnp.float32),
            pltpu.VMEM((rem_pad * dim,), jnp.float32),
            *[pltpu.SemaphoreType.DMA for _ in range(2)],
            *[pltpu.SemaphoreType.DMA for _ in range(2)],
        ],
    )
    def transpose_kernel(tablet_hbm, tail_hbm, out_hbm,
                         va0, va1, vb0, vb1, vat, vbt, sr0, sr1, sw0, sw1):
        va = (va0, va1, vat)
        vb = (vb0, vb1, vbt)
        sr = (sr0, sr1)
        sw = (sw0, sw1)
        wid = lax.axis_index("s") * NC + lax.axis_index("c")
        lane32 = lax.iota(jnp.int32, L) * dim

        def col0(k):
            return pl.multiple_of((k * NW + wid) * CT, CT)

        def read(k, p):
            pltpu.async_copy(
                tablet_hbm.at[:, pl.ds(col0(k), CT)], va[p], sr[p]
            )

        def wait_read(p):
            pltpu.make_async_copy(
                tablet_hbm.at[:, pl.ds(0, CT)], va[p], sr[p]
            ).wait()

        def compute(p, n_cols, batch=4):
            for d0 in range(0, dim, batch):
                def jb(j, cc):
                    vs = [
                        va[p][d0 + u, pl.ds(j * L, L)] for u in range(batch)
                    ]
                    for u in range(batch):
                        plsc.store_scatter(
                            vb[p], [lane32 + (j * (L * dim) + d0 + u)], vs[u]
                        )
                    return cc

                lax.fori_loop(0, n_cols // L, jb, 0)

        def write(k, p):
            pltpu.async_copy(
                vb[p], out_hbm.at[pl.ds(col0(k) * dim, CT * dim)], sw[p]
            )

        def wait_write(p):
            pltpu.make_async_copy(
                vb[p], out_hbm.at[pl.ds(0, CT * dim)], sw[p]
            ).wait()

        read(0, 0)

        def body(i, carry):
            for p in range(2):
                k = 2 * i + p
                wait_read(p)
                read(k + 1, 1 - p)

                @pl.when(i > 0)
                def _ww():
                    wait_write(p)

                compute(p, CT)
                write(k, p)
            return carry

        half = (n_w - 1) // 2
        lax.fori_loop(0, half, body, 0)
        for k in range(2 * half, n_w):
            p = k % 2
            wait_read(p)
            if k + 1 < n_w:
                read(k + 1, 1 - p)
            wait_write(p)
            compute(p, CT)
            write(k, p)
        wait_write(0)
        wait_write(1)

        if rem:
            @pl.when(wid == 0)
            def _tail():
                pltpu.sync_copy(tail_hbm, va[2])
                compute(2, rem, batch=4)
                pltpu.sync_copy(
                    vb[2].at[pl.ds(0, rem * dim)],
                    out_hbm.at[pl.ds((vocab - rem) * dim, rem * dim)],
                )

    return transpose_kernel


@functools.lru_cache(maxsize=None)
def _make_gather(n_flat: int, dim: int):
    assert n_flat % NW == 0
    per_w = n_flat // NW
    assert per_w % (CHUNK * NBUF) == 0
    n_rounds = per_w // (CHUNK * NBUF)

    mesh = plsc.VectorSubcoreMesh(core_axis_name="c", subcore_axis_name="s")

    @functools.partial(
        pl.kernel,
        mesh=mesh,
        compiler_params=pltpu.CompilerParams(use_tc_tiling_on_sc=False),
        out_type=jax.ShapeDtypeStruct((n_flat, 128), jnp.float32),
        scratch_types=[
            pltpu.VMEM((per_w,), jnp.int32),
            *[pltpu.VMEM((CHUNK, dim), jnp.float32) for _ in range(NBUF)],
            *[pltpu.SemaphoreType.DMA for _ in range(NBUF)],
            *[pltpu.SemaphoreType.DMA for _ in range(NBUF)],
        ],
    )
    def gather_kernel(idx_hbm, table_hbm, out_hbm, idx_v, *bufs_and_sems):
        rows = bufs_and_sems[:NBUF]
        sem_g = bufs_and_sems[NBUF:2 * NBUF]
        sem_w = bufs_and_sems[2 * NBUF:3 * NBUF]

        wid = lax.axis_index("s") * NC + lax.axis_index("c")
        base = wid * per_w
        pltpu.sync_copy(idx_hbm.at[pl.ds(base, per_w)], idx_v)

        def body(i, carry):
            rbase = i * (CHUNK * NBUF)
            for b in range(NBUF):
                off = rbase + b * CHUNK

                @pl.when(i > 0)
                def _wait_prev_write():
                    pltpu.make_async_copy(
                        rows[b],
                        out_hbm.at[pl.ds(base + off, CHUNK), pl.ds(0, dim)],
                        sem_w[b],
                    ).wait()

                pltpu.async_copy(
                    table_hbm.at[idx_v.at[pl.ds(off, CHUNK)]], rows[b], sem_g[b]
                )
            for b in range(NBUF):
                off = rbase + b * CHUNK
                pltpu.make_async_copy(
                    table_hbm.at[idx_v.at[pl.ds(off, CHUNK)]], rows[b], sem_g[b]
                ).wait()
                pltpu.async_copy(
                    rows[b],
                    out_hbm.at[pl.ds(base + off, CHUNK), pl.ds(0, dim)],
                    sem_w[b],
                )
            return carry

        lax.fori_loop(0, n_rounds, body, 0)
        for b in range(NBUF):
            pltpu.make_async_copy(
                rows[b], out_hbm.at[pl.ds(base, CHUNK), pl.ds(0, dim)], sem_w[b]
            ).wait()

    return gather_kernel


def kernel(x, table):
    b, t = x.shape
    vocab, dim = table.shape
    flat = jnp.reshape(x, (-1,)).astype(jnp.int32)
    tablet = jnp.transpose(table)
    rem = vocab - (vocab // CT) // NW * NW * CT
    rem_pad = -(-rem // 128) * 128
    tail = jnp.pad(tablet[:, vocab - rem:], ((0, 0), (0, rem_pad - rem)))
    flat_table = _make_transpose(dim, vocab)(tablet, tail)
    tablec = jnp.reshape(flat_table, (vocab, dim))
    out128 = _make_gather(b * t, dim)(flat, tablec)
    return jnp.reshape(out128[:, :dim], (b, t, dim))


# speedup vs baseline: 1.3839x; 1.0129x over previous
import functools

import jax
import jax.numpy as jnp
from jax import lax
from jax.experimental import pallas as pl
from jax.experimental.pallas import tpu as pltpu
from jax.experimental.pallas import tpu_sc as plsc

NC = 2
NS = 16
NW = NC * NS
L = 16

CT = 512
CHUNK = 512
NBUF = 5


@functools.lru_cache(maxsize=None)
def _make_transpose(dim: int, vocab: int):
    n_full = (vocab // CT) // NW * NW
    n_w = n_full // NW
    rem = vocab - n_full * CT
    rem_pad = -(-rem // 128) * 128
    assert n_w >= 3 and rem_pad <= 2 * CT
    mesh = plsc.VectorSubcoreMesh(core_axis_name="c", subcore_axis_name="s")

    @functools.partial(
        pl.kernel,
        mesh=mesh,
        compiler_params=pltpu.CompilerParams(needs_layout_passes=False),
        out_type=jax.ShapeDtypeStruct((vocab * dim,), jnp.float32),
        scratch_types=[
            *[pltpu.VMEM((dim, CT), jnp.float32) for _ in range(2)],
            *[pltpu.VMEM((CT * dim,), jnp.float32) for _ in range(2)],
            pltpu.VMEM((dim, rem_pad), jnp.float32),
            pltpu.VMEM((rem_pad * dim,), jnp.float32),
            *[pltpu.SemaphoreType.DMA for _ in range(2)],
            *[pltpu.SemaphoreType.DMA for _ in range(2)],
        ],
    )
    def transpose_kernel(tablet_hbm, tail_hbm, out_hbm,
                         va0, va1, vb0, vb1, vat, vbt, sr0, sr1, sw0, sw1):
        va = (va0, va1, vat)
        vb = (vb0, vb1, vbt)
        sr = (sr0, sr1)
        sw = (sw0, sw1)
        wid = lax.axis_index("s") * NC + lax.axis_index("c")
        lane32 = lax.iota(jnp.int32, L) * dim

        def col0(k):
            return pl.multiple_of((k * NW + wid) * CT, CT)

        def read(k, p):
            pltpu.async_copy(
                tablet_hbm.at[:, pl.ds(col0(k), CT)], va[p], sr[p]
            )

        def wait_read(p):
            pltpu.make_async_copy(
                tablet_hbm.at[:, pl.ds(0, CT)], va[p], sr[p]
            ).wait()

        def compute(p, n_cols, batch=8):
            for d0 in range(0, dim, batch):
                def jb(j, cc):
                    vs = [
                        va[p][d0 + u, pl.ds(j * L, L)] for u in range(batch)
                    ]
                    for u in range(batch):
                        plsc.store_scatter(
                            vb[p], [lane32 + (j * (L * dim) + d0 + u)], vs[u]
                        )
                    return cc

                lax.fori_loop(0, n_cols // L, jb, 0)

        def write(k, p):
            pltpu.async_copy(
                vb[p], out_hbm.at[pl.ds(col0(k) * dim, CT * dim)], sw[p]
            )

        def wait_write(p):
            pltpu.make_async_copy(
                vb[p], out_hbm.at[pl.ds(0, CT * dim)], sw[p]
            ).wait()

        read(0, 0)

        def body(i, carry):
            for p in range(2):
                k = 2 * i + p
                wait_read(p)
                read(k + 1, 1 - p)

                @pl.when(i > 0)
                def _ww():
                    wait_write(p)

                compute(p, CT)
                write(k, p)
            return carry

        half = (n_w - 1) // 2
        lax.fori_loop(0, half, body, 0)
        for k in range(2 * half, n_w):
            p = k % 2
            wait_read(p)
            if k + 1 < n_w:
                read(k + 1, 1 - p)
            wait_write(p)
            compute(p, CT)
            write(k, p)
        wait_write(0)
        wait_write(1)

        if rem:
            @pl.when(wid == 0)
            def _tail():
                pltpu.sync_copy(tail_hbm, va[2])
                compute(2, rem, batch=4)
                pltpu.sync_copy(
                    vb[2].at[pl.ds(0, rem * dim)],
                    out_hbm.at[pl.ds((vocab - rem) * dim, rem * dim)],
                )

    return transpose_kernel


@functools.lru_cache(maxsize=None)
def _make_gather(n_flat: int, dim: int):
    assert n_flat % NW == 0
    per_w = n_flat // NW
    assert per_w % (CHUNK * NBUF) == 0
    n_rounds = per_w // (CHUNK * NBUF)

    mesh = plsc.VectorSubcoreMesh(core_axis_name="c", subcore_axis_name="s")

    @functools.partial(
        pl.kernel,
        mesh=mesh,
        compiler_params=pltpu.CompilerParams(use_tc_tiling_on_sc=False),
        out_type=jax.ShapeDtypeStruct((n_flat, 128), jnp.float32),
        scratch_types=[
            pltpu.VMEM((per_w,), jnp.int32),
            *[pltpu.VMEM((CHUNK, dim), jnp.float32) for _ in range(NBUF)],
            *[pltpu.SemaphoreType.DMA for _ in range(NBUF)],
            *[pltpu.SemaphoreType.DMA for _ in range(NBUF)],
        ],
    )
    def gather_kernel(idx_hbm, table_hbm, out_hbm, idx_v, *bufs_and_sems):
        rows = bufs_and_sems[:NBUF]
        sem_g = bufs_and_sems[NBUF:2 * NBUF]
        sem_w = bufs_and_sems[2 * NBUF:3 * NBUF]

        wid = lax.axis_index("s") * NC + lax.axis_index("c")
        base = wid * per_w
        pltpu.sync_copy(idx_hbm.at[pl.ds(base, per_w)], idx_v)

        def body(i, carry):
            rbase = i * (CHUNK * NBUF)
            for b in range(NBUF):
                off = rbase + b * CHUNK

                @pl.when(i > 0)
                def _wait_prev_write():
                    pltpu.make_async_copy(
                        rows[b],
                        out_hbm.at[pl.ds(base + off, CHUNK), pl.ds(0, dim)],
                        sem_w[b],
                    ).wait()

                pltpu.async_copy(
                    table_hbm.at[idx_v.at[pl.ds(off, CHUNK)]], rows[b], sem_g[b]
                )
            for b in range(NBUF):
                off = rbase + b * CHUNK
                pltpu.make_async_copy(
                    table_hbm.at[idx_v.at[pl.ds(off, CHUNK)]], rows[b], sem_g[b]
                ).wait()
                pltpu.async_copy(
                    rows[b],
                    out_hbm.at[pl.ds(base + off, CHUNK), pl.ds(0, dim)],
                    sem_w[b],
                )
            return carry

        lax.fori_loop(0, n_rounds, body, 0)
        for b in range(NBUF):
            pltpu.make_async_copy(
                rows[b], out_hbm.at[pl.ds(base, CHUNK), pl.ds(0, dim)], sem_w[b]
            ).wait()

    return gather_kernel


def kernel(x, table):
    b, t = x.shape
    vocab, dim = table.shape
    flat = jnp.reshape(x, (-1,)).astype(jnp.int32)
    tablet = jnp.transpose(table)
    rem = vocab - (vocab // CT) // NW * NW * CT
    rem_pad = -(-rem // 128) * 128
    tail = jnp.pad(tablet[:, vocab - rem:], ((0, 0), (0, rem_pad - rem)))
    flat_table = _make_transpose(dim, vocab)(tablet, tail)
    tablec = jnp.reshape(flat_table, (vocab, dim))
    out128 = _make_gather(b * t, dim)(flat, tablec)
    return jnp.reshape(out128[:, :dim], (b, t, dim))


# speedup vs baseline: 1.3845x; 1.0004x over previous
import functools

import jax
import jax.numpy as jnp
from jax import lax
from jax.experimental import pallas as pl
from jax.experimental.pallas import tpu as pltpu
from jax.experimental.pallas import tpu_sc as plsc

NC = 2
NS = 16
NW = NC * NS
L = 16

CT = 512
CHUNK = 512
NBUF = 5


@functools.lru_cache(maxsize=None)
def _make_transpose(dim: int, vocab: int):
    n_full = (vocab // CT) // NW * NW
    n_w = n_full // NW
    rem = vocab - n_full * CT
    rem_pad = -(-rem // 128) * 128
    assert n_w >= 3 and rem_pad <= 2 * CT
    mesh = plsc.VectorSubcoreMesh(core_axis_name="c", subcore_axis_name="s")

    @functools.partial(
        pl.kernel,
        mesh=mesh,
        compiler_params=pltpu.CompilerParams(needs_layout_passes=False),
        out_type=jax.ShapeDtypeStruct((vocab * dim,), jnp.float32),
        scratch_types=[
            *[pltpu.VMEM((dim, CT), jnp.float32) for _ in range(2)],
            *[pltpu.VMEM((CT * dim,), jnp.float32) for _ in range(2)],
            pltpu.VMEM((dim, rem_pad), jnp.float32),
            pltpu.VMEM((rem_pad * dim,), jnp.float32),
            *[pltpu.SemaphoreType.DMA for _ in range(2)],
            *[pltpu.SemaphoreType.DMA for _ in range(2)],
        ],
    )
    def transpose_kernel(tablet_hbm, tail_hbm, out_hbm,
                         va0, va1, vb0, vb1, vat, vbt, sr0, sr1, sw0, sw1):
        va = (va0, va1, vat)
        vb = (vb0, vb1, vbt)
        sr = (sr0, sr1)
        sw = (sw0, sw1)
        wid = lax.axis_index("s") * NC + lax.axis_index("c")
        lane32 = lax.iota(jnp.int32, L) * dim

        def col0(k):
            return pl.multiple_of((k * NW + wid) * CT, CT)

        def read(k, p):
            pltpu.async_copy(
                tablet_hbm.at[:, pl.ds(col0(k), CT)], va[p], sr[p]
            )

        def wait_read(p):
            pltpu.make_async_copy(
                tablet_hbm.at[:, pl.ds(0, CT)], va[p], sr[p]
            ).wait()

        def compute(p, n_cols, batch=16):
            for d0 in range(0, dim, batch):
                def jb(j, cc):
                    vs = [
                        va[p][d0 + u, pl.ds(j * L, L)] for u in range(batch)
                    ]
                    for u in range(batch):
                        plsc.store_scatter(
                            vb[p], [lane32 + (j * (L * dim) + d0 + u)], vs[u]
                        )
                    return cc

                lax.fori_loop(0, n_cols // L, jb, 0)

        def write(k, p):
            pltpu.async_copy(
                vb[p], out_hbm.at[pl.ds(col0(k) * dim, CT * dim)], sw[p]
            )

        def wait_write(p):
            pltpu.make_async_copy(
                vb[p], out_hbm.at[pl.ds(0, CT * dim)], sw[p]
            ).wait()

        read(0, 0)

        def body(i, carry):
            for p in range(2):
                k = 2 * i + p
                wait_read(p)
                read(k + 1, 1 - p)

                @pl.when(i > 0)
                def _ww():
                    wait_write(p)

                compute(p, CT)
                write(k, p)
            return carry

        half = (n_w - 1) // 2
        lax.fori_loop(0, half, body, 0)
        for k in range(2 * half, n_w):
            p = k % 2
            wait_read(p)
            if k + 1 < n_w:
                read(k + 1, 1 - p)
            wait_write(p)
            compute(p, CT)
            write(k, p)
        wait_write(0)
        wait_write(1)

        if rem:
            @pl.when(wid == 0)
            def _tail():
                pltpu.sync_copy(tail_hbm, va[2])
                compute(2, rem, batch=4)
                pltpu.sync_copy(
                    vb[2].at[pl.ds(0, rem * dim)],
                    out_hbm.at[pl.ds((vocab - rem) * dim, rem * dim)],
                )

    return transpose_kernel


@functools.lru_cache(maxsize=None)
def _make_gather(n_flat: int, dim: int):
    assert n_flat % NW == 0
    per_w = n_flat // NW
    assert per_w % (CHUNK * NBUF) == 0
    n_rounds = per_w // (CHUNK * NBUF)

    mesh = plsc.VectorSubcoreMesh(core_axis_name="c", subcore_axis_name="s")

    @functools.partial(
        pl.kernel,
        mesh=mesh,
        compiler_params=pltpu.CompilerParams(use_tc_tiling_on_sc=False),
        out_type=jax.ShapeDtypeStruct((n_flat, 128), jnp.float32),
        scratch_types=[
            pltpu.VMEM((per_w,), jnp.int32),
            *[pltpu.VMEM((CHUNK, dim), jnp.float32) for _ in range(NBUF)],
            *[pltpu.SemaphoreType.DMA for _ in range(NBUF)],
            *[pltpu.SemaphoreType.DMA for _ in range(NBUF)],
        ],
    )
    def gather_kernel(idx_hbm, table_hbm, out_hbm, idx_v, *bufs_and_sems):
        rows = bufs_and_sems[:NBUF]
        sem_g = bufs_and_sems[NBUF:2 * NBUF]
        sem_w = bufs_and_sems[2 * NBUF:3 * NBUF]

        wid = lax.axis_index("s") * NC + lax.axis_index("c")
        base = wid * per_w
        pltpu.sync_copy(idx_hbm.at[pl.ds(base, per_w)], idx_v)

        def body(i, carry):
            rbase = i * (CHUNK * NBUF)
            for b in range(NBUF):
                off = rbase + b * CHUNK

                @pl.when(i > 0)
                def _wait_prev_write():
                    pltpu.make_async_copy(
                        rows[b],
                        out_hbm.at[pl.ds(base + off, CHUNK), pl.ds(0, dim)],
                        sem_w[b],
                    ).wait()

                pltpu.async_copy(
                    table_hbm.at[idx_v.at[pl.ds(off, CHUNK)]], rows[b], sem_g[b]
                )
            for b in range(NBUF):
                off = rbase + b * CHUNK
                pltpu.make_async_copy(
                    table_hbm.at[idx_v.at[pl.ds(off, CHUNK)]], rows[b], sem_g[b]
                ).wait()
                pltpu.async_copy(
                    rows[b],
                    out_hbm.at[pl.ds(base + off, CHUNK), pl.ds(0, dim)],
                    sem_w[b],
                )
            return carry

        lax.fori_loop(0, n_rounds, body, 0)
        for b in range(NBUF):
            pltpu.make_async_copy(
                rows[b], out_hbm.at[pl.ds(base, CHUNK), pl.ds(0, dim)], sem_w[b]
            ).wait()

    return gather_kernel


def kernel(x, table):
    b, t = x.shape
    vocab, dim = table.shape
    flat = jnp.reshape(x, (-1,)).astype(jnp.int32)
    tablet = jnp.transpose(table)
    rem = vocab - (vocab // CT) // NW * NW * CT
    rem_pad = -(-rem // 128) * 128
    tail = jnp.pad(tablet[:, vocab - rem:], ((0, 0), (0, rem_pad - rem)))
    flat_table = _make_transpose(dim, vocab)(tablet, tail)
    tablec = jnp.reshape(flat_table, (vocab, dim))
    out128 = _make_gather(b * t, dim)(flat, tablec)
    return jnp.reshape(out128[:, :dim], (b, t, dim))
